# Initial kernel scaffold; baseline (speedup 1.0000x reference)
import jax
import jax.numpy as jnp
from jax.experimental import pallas as pl


def kernel(week_ids, table, W, b):
    raise NotImplementedError("write your pallas kernel here")


# speedup vs baseline: 3.1124x; 3.1124x over previous
import functools

import jax
import jax.numpy as jnp
from jax import lax
from jax.experimental import pallas as pl
from jax.experimental.pallas import tpu as pltpu
from jax.experimental.pallas import tpu_sc as plsc

H = 64
VPAD = 64


def _proj_body(table_ref, w_ref, b_ref, out_ref):
    out_ref[...] = (
        lax.dot_general(
            table_ref[...], w_ref[...],
            (((1,), (1,)), ((), ())),
            preferred_element_type=jnp.float32,
        )
        + b_ref[...]
    )


@functools.partial(jax.jit, static_argnums=(2, 3))
def _gather_call(proj, ids, b_per_w, chunk):
    mesh = plsc.VectorSubcoreMesh(core_axis_name="c", subcore_axis_name="s")
    num_chunks = b_per_w // chunk
    B = ids.shape[0]

    @functools.partial(
        pl.kernel,
        mesh=mesh,
        out_type=jax.ShapeDtypeStruct((B, H), jnp.float32),
        scratch_types=[
            pltpu.VMEM((chunk,), jnp.int32),
            pltpu.VMEM((chunk, H), jnp.float32),
            pltpu.SemaphoreType.DMA,
        ],
        compiler_params=pltpu.CompilerParams(use_tc_tiling_on_sc=False),
    )
    def k(proj_hbm, idx_hbm, out_hbm, idx_v, rows_v, sem):
        wid = lax.axis_index("s") * 2 + lax.axis_index("c")
        base = wid * b_per_w

        def body(g, carry):
            off = base + g * chunk
            pltpu.sync_copy(idx_hbm.at[pl.ds(off, chunk)], idx_v)
            pltpu.async_copy(proj_hbm.at[idx_v], rows_v, sem).wait()
            pltpu.sync_copy(rows_v, out_hbm.at[pl.ds(off, chunk)])
            return carry

        lax.fori_loop(0, num_chunks, body, 0)

    return k(proj, ids)


@jax.jit
def kernel(week_ids, table, W, b):
    Bseq, L = week_ids.shape
    ids = week_ids.reshape(-1).astype(jnp.int32)

    table_pad = jnp.zeros((VPAD, H), jnp.float32).at[: table.shape[0]].set(table)
    proj = pl.pallas_call(
        _proj_body,
        out_shape=jax.ShapeDtypeStruct((VPAD, H), jnp.float32),
    )(table_pad, W, b.reshape(1, H))

    B = ids.shape[0]
    b_per_w = B // 32
    out = _gather_call(proj, ids, b_per_w, 512)
    return out.reshape(Bseq, L, H)
